# Initial kernel scaffold; baseline (speedup 1.0000x reference)
import jax
import jax.numpy as jnp
from jax.experimental import pallas as pl


def kernel(node_feat, edge_index, graph_ids, W_n2l, W_conv, add_W1, add_b1, add_W2, add_b2):
    raise NotImplementedError("write your pallas kernel here")


# speedup vs baseline: 3.1706x; 3.1706x over previous
import functools

import jax
import jax.numpy as jnp
from jax import lax
from jax.experimental import pallas as pl
from jax.experimental.pallas import tpu as pltpu
from jax.experimental.pallas import tpu_sc as plsc

N = 50000
E = 1600000
D = 64
H = 128
B = 16
MAX_LV = 3

NC = 2
NS = 16
NW = NC * NS
PASSES = D // NW
LANES = 16
CHUNK = 4000
NCH = E // CHUNK
BLK = 512

_mesh = plsc.VectorSubcoreMesh(core_axis_name="c", subcore_axis_name="s")


@functools.partial(
    pl.kernel,
    out_type=jax.ShapeDtypeStruct((D, N), jnp.float32),
    mesh=_mesh,
    compiler_params=pltpu.CompilerParams(needs_layout_passes=False),
    scratch_types=[
        pltpu.VMEM((N,), jnp.float32),
        pltpu.VMEM((N,), jnp.float32),
        pltpu.VMEM((CHUNK,), jnp.int32),
        pltpu.VMEM((CHUNK,), jnp.int32),
        pltpu.VMEM((CHUNK,), jnp.int32),
        pltpu.VMEM((CHUNK,), jnp.int32),
        pltpu.SemaphoreType.DMA,
        pltpu.SemaphoreType.DMA,
    ],
)
def _seg_sum_T(h_hbm, src_hbm, dst_hbm, out_hbm, tab, acc, s0, d0, s1, d1, sem0, sem1):
    wid = lax.axis_index("s") * NC + lax.axis_index("c")
    sems = (sem0, sem1)
    sbufs = (s0, s1)
    dbufs = (d0, d1)

    def issue(k, b):
        pltpu.async_copy(src_hbm.at[pl.ds(k * CHUNK, CHUNK)], sbufs[b], sems[b])
        pltpu.async_copy(dst_hbm.at[pl.ds(k * CHUNK, CHUNK)], dbufs[b], sems[b])

    def drain(b):
        pltpu.make_async_copy(src_hbm.at[pl.ds(0, CHUNK)], sbufs[b], sems[b]).wait()
        pltpu.make_async_copy(dst_hbm.at[pl.ds(0, CHUNK)], dbufs[b], sems[b]).wait()

    for p in range(PASSES):
        col = wid + NW * p
        pltpu.sync_copy(h_hbm.at[col], tab)

        def zero_body(i, carry):
            acc[pl.ds(i * LANES, LANES)] = jnp.zeros((LANES,), jnp.float32)
            return carry

        lax.fori_loop(0, N // LANES, zero_body, 0)

        issue(0, 0)
        issue(1, 1)

        def chunk_pair(g, carry):
            for b in range(2):
                k = 2 * g + b
                drain(b)

                def inner(j, c):
                    s = sbufs[b][pl.ds(j * LANES, LANES)]
                    dv = dbufs[b][pl.ds(j * LANES, LANES)]
                    vals = plsc.load_gather(tab, [s])
                    plsc.addupdate_scatter(acc, [dv], vals)
                    return c

                lax.fori_loop(0, CHUNK // LANES, inner, 0)

                @pl.when(k + 2 < NCH)
                def _():
                    issue(k + 2, b)

            return carry

        lax.fori_loop(0, NCH // 2, chunk_pair, 0)

        pltpu.sync_copy(acc, out_hbm.at[col])


def _prelude_body(nfT_ref, wT_ref, msg_ref, h_ref):
    m = jnp.dot(wT_ref[...], nfT_ref[...], preferred_element_type=jnp.float32)
    msg_ref[...] = m
    h_ref[...] = jnp.maximum(m, 0.0)


_prelude = pl.pallas_call(
    _prelude_body,
    grid=(pl.cdiv(N, BLK),),
    in_specs=[
        pl.BlockSpec((2, BLK), lambda i: (0, i)),
        pl.BlockSpec((D, 2), lambda i: (0, 0)),
    ],
    out_specs=[
        pl.BlockSpec((D, BLK), lambda i: (0, i)),
        pl.BlockSpec((D, BLK), lambda i: (0, i)),
    ],
    out_shape=[
        jax.ShapeDtypeStruct((D, N), jnp.float32),
        jax.ShapeDtypeStruct((D, N), jnp.float32),
    ],
)


def _level_body(poolT_ref, wT_ref, msgT_ref, h_ref):
    m = jnp.dot(wT_ref[...], poolT_ref[...], preferred_element_type=jnp.float32)
    h_ref[...] = jnp.maximum(m + msgT_ref[...], 0.0)


_level = pl.pallas_call(
    _level_body,
    grid=(pl.cdiv(N, BLK),),
    in_specs=[
        pl.BlockSpec((D, BLK), lambda i: (0, i)),
        pl.BlockSpec((D, D), lambda i: (0, 0)),
        pl.BlockSpec((D, BLK), lambda i: (0, i)),
    ],
    out_specs=pl.BlockSpec((D, BLK), lambda i: (0, i)),
    out_shape=jax.ShapeDtypeStruct((D, N), jnp.float32),
)


def _pool_body(hT_ref, gid_ref, out_ref):
    i = pl.program_id(0)
    ids = gid_ref[0, :]
    row = lax.broadcasted_iota(jnp.int32, (BLK, B), 0)
    valid = (i * BLK + row) < N
    gcol = lax.broadcasted_iota(jnp.int32, (BLK, B), 1)
    onehot = jnp.where((ids[:, None] == gcol) & valid, 1.0, 0.0)
    part = jnp.dot(hT_ref[...], onehot, preferred_element_type=jnp.float32)

    @pl.when(i == 0)
    def _():
        out_ref[...] = jnp.zeros_like(out_ref)

    out_ref[...] += part


_pool = pl.pallas_call(
    _pool_body,
    grid=(pl.cdiv(N, BLK),),
    in_specs=[
        pl.BlockSpec((D, BLK), lambda i: (0, i)),
        pl.BlockSpec((1, BLK), lambda i: (0, i)),
    ],
    out_specs=pl.BlockSpec((D, B), lambda i: (0, 0)),
    out_shape=jax.ShapeDtypeStruct((D, B), jnp.float32),
)


def _head_body(xT_ref, w1T_ref, b1_ref, w2T_ref, b2_ref, out_ref):
    hid = jnp.dot(w1T_ref[...], xT_ref[...], preferred_element_type=jnp.float32)
    hid = jnp.maximum(hid + b1_ref[...], 0.0)
    out_ref[...] = jnp.dot(w2T_ref[...], hid, preferred_element_type=jnp.float32) + b2_ref[...]


def _make_head(ncols, blk):
    return pl.pallas_call(
        _head_body,
        grid=(pl.cdiv(ncols, blk),),
        in_specs=[
            pl.BlockSpec((D, blk), lambda i: (0, i)),
            pl.BlockSpec((H, D), lambda i: (0, 0)),
            pl.BlockSpec((H, 1), lambda i: (0, 0)),
            pl.BlockSpec((1, H), lambda i: (0, 0)),
            pl.BlockSpec((1, 1), lambda i: (0, 0)),
        ],
        out_specs=pl.BlockSpec((1, blk), lambda i: (0, i)),
        out_shape=jax.ShapeDtypeStruct((1, ncols), jnp.float32),
    )


_head_nodes = _make_head(N, BLK)
_head_graphs = _make_head(B, B)


def kernel(node_feat, edge_index, graph_ids, W_n2l, W_conv, add_W1, add_b1, add_W2, add_b2):
    ei = edge_index.astype(jnp.int32)
    nfT = node_feat.T
    msgT, hT = _prelude(nfT, W_n2l.T)
    wcT = W_conv.T
    for _ in range(MAX_LV):
        poolT = _seg_sum_T(hT, ei[0], ei[1])
        hT = _level(poolT, wcT, msgT)
    geT = _pool(hT, graph_ids.astype(jnp.int32).reshape(1, N))
    w1T = add_W1.T
    b1c = add_b1.reshape(H, 1)
    w2T = add_W2.T
    b2c = add_b2.reshape(1, 1)
    raw_n = _head_nodes(hT, w1T, b1c, w2T, b2c)
    raw_g = _head_graphs(geT, w1T, b1c, w2T, b2c)
    return jnp.concatenate([raw_n[0], raw_g[0]])[:, None]


# speedup vs baseline: 7.6914x; 2.4259x over previous
import functools

import jax
import jax.numpy as jnp
from jax import lax
from jax.experimental import pallas as pl
from jax.experimental.pallas import tpu as pltpu
from jax.experimental.pallas import tpu_sc as plsc

N = 50000
E = 1600000
D = 64
H = 128
B = 16
MAX_LV = 3

NC = 2
NS = 16
NW = NC * NS
PASSES = D // NW
LANES = 16
CHUNK = 4000
NCH = E // CHUNK
BLK = 512

_mesh = plsc.VectorSubcoreMesh(core_axis_name="c", subcore_axis_name="s")


@functools.partial(
    pl.kernel,
    out_type=jax.ShapeDtypeStruct((D, N), jnp.float32),
    mesh=_mesh,
    compiler_params=pltpu.CompilerParams(needs_layout_passes=False),
    scratch_types=[
        pltpu.VMEM((N,), jnp.float32),
        pltpu.VMEM((N,), jnp.float32),
        pltpu.VMEM((CHUNK,), jnp.int32),
        pltpu.VMEM((CHUNK,), jnp.int32),
        pltpu.VMEM((CHUNK,), jnp.int32),
        pltpu.VMEM((CHUNK,), jnp.int32),
        pltpu.SemaphoreType.DMA,
        pltpu.SemaphoreType.DMA,
    ],
)
def _seg_sum_T(h_hbm, src_hbm, dst_hbm, out_hbm, tab, acc, s0, d0, s1, d1, sem0, sem1):
    wid = lax.axis_index("s") * NC + lax.axis_index("c")
    sems = (sem0, sem1)
    sbufs = (s0, s1)
    dbufs = (d0, d1)

    def issue(k, b):
        pltpu.async_copy(src_hbm.at[pl.ds(k * CHUNK, CHUNK)], sbufs[b], sems[b])
        pltpu.async_copy(dst_hbm.at[pl.ds(k * CHUNK, CHUNK)], dbufs[b], sems[b])

    def drain(b):
        pltpu.make_async_copy(src_hbm.at[pl.ds(0, CHUNK)], sbufs[b], sems[b]).wait()
        pltpu.make_async_copy(dst_hbm.at[pl.ds(0, CHUNK)], dbufs[b], sems[b]).wait()

    for p in range(PASSES):
        col = wid + NW * p
        pltpu.sync_copy(h_hbm.at[col], tab)

        @plsc.parallel_loop(0, N, step=LANES, unroll=8)
        def _(i):
            acc[pl.ds(i, LANES)] = jnp.zeros((LANES,), jnp.float32)

        issue(0, 0)
        issue(1, 1)

        def chunk_pair(g, carry):
            for b in range(2):
                k = 2 * g + b
                drain(b)

                @plsc.parallel_loop(0, CHUNK, step=LANES, unroll=8)
                def _(j):
                    s = sbufs[b][pl.ds(j, LANES)]
                    dv = dbufs[b][pl.ds(j, LANES)]
                    vals = plsc.load_gather(tab, [s])
                    plsc.addupdate_scatter(acc, [dv], vals)

                @pl.when(k + 2 < NCH)
                def _():
                    issue(k + 2, b)

            return carry

        lax.fori_loop(0, NCH // 2, chunk_pair, 0)

        pltpu.sync_copy(acc, out_hbm.at[col])


def _prelude_body(nfT_ref, wT_ref, msg_ref, h_ref):
    m = jnp.dot(wT_ref[...], nfT_ref[...], preferred_element_type=jnp.float32)
    msg_ref[...] = m
    h_ref[...] = jnp.maximum(m, 0.0)


_prelude = pl.pallas_call(
    _prelude_body,
    grid=(pl.cdiv(N, BLK),),
    in_specs=[
        pl.BlockSpec((2, BLK), lambda i: (0, i)),
        pl.BlockSpec((D, 2), lambda i: (0, 0)),
    ],
    out_specs=[
        pl.BlockSpec((D, BLK), lambda i: (0, i)),
        pl.BlockSpec((D, BLK), lambda i: (0, i)),
    ],
    out_shape=[
        jax.ShapeDtypeStruct((D, N), jnp.float32),
        jax.ShapeDtypeStruct((D, N), jnp.float32),
    ],
)


def _level_body(poolT_ref, wT_ref, msgT_ref, h_ref):
    m = jnp.dot(wT_ref[...], poolT_ref[...], preferred_element_type=jnp.float32)
    h_ref[...] = jnp.maximum(m + msgT_ref[...], 0.0)


_level = pl.pallas_call(
    _level_body,
    grid=(pl.cdiv(N, BLK),),
    in_specs=[
        pl.BlockSpec((D, BLK), lambda i: (0, i)),
        pl.BlockSpec((D, D), lambda i: (0, 0)),
        pl.BlockSpec((D, BLK), lambda i: (0, i)),
    ],
    out_specs=pl.BlockSpec((D, BLK), lambda i: (0, i)),
    out_shape=jax.ShapeDtypeStruct((D, N), jnp.float32),
)


def _pool_body(hT_ref, gid_ref, out_ref):
    i = pl.program_id(0)
    ids = gid_ref[0, :]
    row = lax.broadcasted_iota(jnp.int32, (BLK, B), 0)
    valid = (i * BLK + row) < N
    gcol = lax.broadcasted_iota(jnp.int32, (BLK, B), 1)
    onehot = jnp.where((ids[:, None] == gcol) & valid, 1.0, 0.0)
    part = jnp.dot(hT_ref[...], onehot, preferred_element_type=jnp.float32)

    @pl.when(i == 0)
    def _():
        out_ref[...] = jnp.zeros_like(out_ref)

    out_ref[...] += part


_pool = pl.pallas_call(
    _pool_body,
    grid=(pl.cdiv(N, BLK),),
    in_specs=[
        pl.BlockSpec((D, BLK), lambda i: (0, i)),
        pl.BlockSpec((1, BLK), lambda i: (0, i)),
    ],
    out_specs=pl.BlockSpec((D, B), lambda i: (0, 0)),
    out_shape=jax.ShapeDtypeStruct((D, B), jnp.float32),
)


def _head_body(xT_ref, w1T_ref, b1_ref, w2T_ref, b2_ref, out_ref):
    hid = jnp.dot(w1T_ref[...], xT_ref[...], preferred_element_type=jnp.float32)
    hid = jnp.maximum(hid + b1_ref[...], 0.0)
    out_ref[...] = jnp.dot(w2T_ref[...], hid, preferred_element_type=jnp.float32) + b2_ref[...]


def _make_head(ncols, blk):
    return pl.pallas_call(
        _head_body,
        grid=(pl.cdiv(ncols, blk),),
        in_specs=[
            pl.BlockSpec((D, blk), lambda i: (0, i)),
            pl.BlockSpec((H, D), lambda i: (0, 0)),
            pl.BlockSpec((H, 1), lambda i: (0, 0)),
            pl.BlockSpec((1, H), lambda i: (0, 0)),
            pl.BlockSpec((1, 1), lambda i: (0, 0)),
        ],
        out_specs=pl.BlockSpec((1, blk), lambda i: (0, i)),
        out_shape=jax.ShapeDtypeStruct((1, ncols), jnp.float32),
    )


_head_nodes = _make_head(N, BLK)
_head_graphs = _make_head(B, B)


def kernel(node_feat, edge_index, graph_ids, W_n2l, W_conv, add_W1, add_b1, add_W2, add_b2):
    ei = edge_index.astype(jnp.int32)
    nfT = node_feat.T
    msgT, hT = _prelude(nfT, W_n2l.T)
    wcT = W_conv.T
    for _ in range(MAX_LV):
        poolT = _seg_sum_T(hT, ei[0], ei[1])
        hT = _level(poolT, wcT, msgT)
    geT = _pool(hT, graph_ids.astype(jnp.int32).reshape(1, N))
    w1T = add_W1.T
    b1c = add_b1.reshape(H, 1)
    w2T = add_W2.T
    b2c = add_b2.reshape(1, 1)
    raw_n = _head_nodes(hT, w1T, b1c, w2T, b2c)
    raw_g = _head_graphs(geT, w1T, b1c, w2T, b2c)
    return jnp.concatenate([raw_n[0], raw_g[0]])[:, None]


# speedup vs baseline: 8.2564x; 1.0735x over previous
import functools

import jax
import jax.numpy as jnp
from jax import lax
from jax.experimental import pallas as pl
from jax.experimental.pallas import tpu as pltpu
from jax.experimental.pallas import tpu_sc as plsc

N = 50000
E = 1600000
D = 64
H = 128
B = 16
MAX_LV = 3

NC = 2
NS = 16
NW = NC * NS
PASSES = D // NW
LANES = 16
CHUNK = 6400
NCH = E // CHUNK
BLK = 512

_mesh = plsc.VectorSubcoreMesh(core_axis_name="c", subcore_axis_name="s")


@functools.partial(
    pl.kernel,
    out_type=jax.ShapeDtypeStruct((D, N), jnp.float32),
    mesh=_mesh,
    compiler_params=pltpu.CompilerParams(needs_layout_passes=False),
    scratch_types=[
        pltpu.VMEM((N,), jnp.float32),
        pltpu.VMEM((N,), jnp.float32),
        pltpu.VMEM((CHUNK,), jnp.int32),
        pltpu.VMEM((CHUNK,), jnp.int32),
        pltpu.VMEM((CHUNK,), jnp.int32),
        pltpu.VMEM((CHUNK,), jnp.int32),
        pltpu.SemaphoreType.DMA,
        pltpu.SemaphoreType.DMA,
    ],
)
def _seg_sum_T(h_hbm, src_hbm, dst_hbm, out_hbm, tab, acc, s0, d0, s1, d1, sem0, sem1):
    wid = lax.axis_index("s") * NC + lax.axis_index("c")
    sems = (sem0, sem1)
    sbufs = (s0, s1)
    dbufs = (d0, d1)

    def issue(k, b):
        pltpu.async_copy(src_hbm.at[pl.ds(k * CHUNK, CHUNK)], sbufs[b], sems[b])
        pltpu.async_copy(dst_hbm.at[pl.ds(k * CHUNK, CHUNK)], dbufs[b], sems[b])

    def drain(b):
        pltpu.make_async_copy(src_hbm.at[pl.ds(0, CHUNK)], sbufs[b], sems[b]).wait()
        pltpu.make_async_copy(dst_hbm.at[pl.ds(0, CHUNK)], dbufs[b], sems[b]).wait()

    for p in range(PASSES):
        col = wid + NW * p
        pltpu.sync_copy(h_hbm.at[col], tab)

        @plsc.parallel_loop(0, N, step=LANES, unroll=16)
        def _(i):
            acc[pl.ds(i, LANES)] = jnp.zeros((LANES,), jnp.float32)

        issue(0, 0)
        issue(1, 1)

        def chunk_pair(g, carry):
            for b in range(2):
                k = 2 * g + b
                drain(b)

                @plsc.parallel_loop(0, CHUNK, step=LANES, unroll=16)
                def _(j):
                    s = sbufs[b][pl.ds(j, LANES)]
                    dv = dbufs[b][pl.ds(j, LANES)]
                    vals = plsc.load_gather(tab, [s])
                    plsc.addupdate_scatter(acc, [dv], vals)

                @pl.when(k + 2 < NCH)
                def _():
                    issue(k + 2, b)

            return carry

        lax.fori_loop(0, NCH // 2, chunk_pair, 0)

        pltpu.sync_copy(acc, out_hbm.at[col])


def _prelude_body(nfT_ref, wT_ref, msg_ref, h_ref):
    m = jnp.dot(wT_ref[...], nfT_ref[...], preferred_element_type=jnp.float32)
    msg_ref[...] = m
    h_ref[...] = jnp.maximum(m, 0.0)


_prelude = pl.pallas_call(
    _prelude_body,
    grid=(pl.cdiv(N, BLK),),
    in_specs=[
        pl.BlockSpec((2, BLK), lambda i: (0, i)),
        pl.BlockSpec((D, 2), lambda i: (0, 0)),
    ],
    out_specs=[
        pl.BlockSpec((D, BLK), lambda i: (0, i)),
        pl.BlockSpec((D, BLK), lambda i: (0, i)),
    ],
    out_shape=[
        jax.ShapeDtypeStruct((D, N), jnp.float32),
        jax.ShapeDtypeStruct((D, N), jnp.float32),
    ],
)


def _level_body(poolT_ref, wT_ref, msgT_ref, h_ref):
    m = jnp.dot(wT_ref[...], poolT_ref[...], preferred_element_type=jnp.float32)
    h_ref[...] = jnp.maximum(m + msgT_ref[...], 0.0)


_level = pl.pallas_call(
    _level_body,
    grid=(pl.cdiv(N, BLK),),
    in_specs=[
        pl.BlockSpec((D, BLK), lambda i: (0, i)),
        pl.BlockSpec((D, D), lambda i: (0, 0)),
        pl.BlockSpec((D, BLK), lambda i: (0, i)),
    ],
    out_specs=pl.BlockSpec((D, BLK), lambda i: (0, i)),
    out_shape=jax.ShapeDtypeStruct((D, N), jnp.float32),
)


def _pool_body(hT_ref, gid_ref, out_ref):
    i = pl.program_id(0)
    ids = gid_ref[0, :]
    row = lax.broadcasted_iota(jnp.int32, (BLK, B), 0)
    valid = (i * BLK + row) < N
    gcol = lax.broadcasted_iota(jnp.int32, (BLK, B), 1)
    onehot = jnp.where((ids[:, None] == gcol) & valid, 1.0, 0.0)
    part = jnp.dot(hT_ref[...], onehot, preferred_element_type=jnp.float32)

    @pl.when(i == 0)
    def _():
        out_ref[...] = jnp.zeros_like(out_ref)

    out_ref[...] += part


_pool = pl.pallas_call(
    _pool_body,
    grid=(pl.cdiv(N, BLK),),
    in_specs=[
        pl.BlockSpec((D, BLK), lambda i: (0, i)),
        pl.BlockSpec((1, BLK), lambda i: (0, i)),
    ],
    out_specs=pl.BlockSpec((D, B), lambda i: (0, 0)),
    out_shape=jax.ShapeDtypeStruct((D, B), jnp.float32),
)


def _head_body(xT_ref, w1T_ref, b1_ref, w2T_ref, b2_ref, out_ref):
    hid = jnp.dot(w1T_ref[...], xT_ref[...], preferred_element_type=jnp.float32)
    hid = jnp.maximum(hid + b1_ref[...], 0.0)
    out_ref[...] = jnp.dot(w2T_ref[...], hid, preferred_element_type=jnp.float32) + b2_ref[...]


def _make_head(ncols, blk):
    return pl.pallas_call(
        _head_body,
        grid=(pl.cdiv(ncols, blk),),
        in_specs=[
            pl.BlockSpec((D, blk), lambda i: (0, i)),
            pl.BlockSpec((H, D), lambda i: (0, 0)),
            pl.BlockSpec((H, 1), lambda i: (0, 0)),
            pl.BlockSpec((1, H), lambda i: (0, 0)),
            pl.BlockSpec((1, 1), lambda i: (0, 0)),
        ],
        out_specs=pl.BlockSpec((1, blk), lambda i: (0, i)),
        out_shape=jax.ShapeDtypeStruct((1, ncols), jnp.float32),
    )


_head_nodes = _make_head(N, BLK)
_head_graphs = _make_head(B, B)


def kernel(node_feat, edge_index, graph_ids, W_n2l, W_conv, add_W1, add_b1, add_W2, add_b2):
    ei = edge_index.astype(jnp.int32)
    nfT = node_feat.T
    msgT, hT = _prelude(nfT, W_n2l.T)
    wcT = W_conv.T
    for _ in range(MAX_LV):
        poolT = _seg_sum_T(hT, ei[0], ei[1])
        hT = _level(poolT, wcT, msgT)
    geT = _pool(hT, graph_ids.astype(jnp.int32).reshape(1, N))
    w1T = add_W1.T
    b1c = add_b1.reshape(H, 1)
    w2T = add_W2.T
    b2c = add_b2.reshape(1, 1)
    raw_n = _head_nodes(hT, w1T, b1c, w2T, b2c)
    raw_g = _head_graphs(geT, w1T, b1c, w2T, b2c)
    return jnp.concatenate([raw_n[0], raw_g[0]])[:, None]
